# baseline (speedup 1.0000x reference)
import jax
import jax.numpy as jnp
from jax import lax
from jax.experimental import pallas as pl
from jax.experimental.pallas import tpu as pltpu

NUM_CANDIDATES = 1000000
D = 64
K = 100
CB = 8192
R = CB // 128
GRID = -(-NUM_CANDIDATES // CB)
NSUB = 62 * 128
NC = 62
NEG = float("-inf")
BIG = 1 << 30


def _lane_iota():
    return lax.broadcasted_iota(jnp.int32, (1, 128), 1)


def _body(sp, u0, u1, g0, g1, a0, a1, cand, out_s, out_i, S, RM, L1, L2):
    j = pl.program_id(0)

    @pl.when(j == 0)
    def _init_tail():
        S[:, GRID * R:, :] = jnp.full((2, NSUB - GRID * R, 128), NEG, jnp.float32)
        RM[:, GRID * R:, :] = jnp.full((2, NSUB - GRID * R, 1), NEG, jnp.float32)

    q = jnp.concatenate(
        [u0[0] + g0[0] + a0[0], u1[0] + g1[0] + a1[0]], axis=0
    )
    s = lax.dot_general(
        q, cand[...], (((1,), (1,)), ((), ())),
        preferred_element_type=jnp.float32,
    )
    gpos = j * CB + lax.broadcasted_iota(jnp.int32, (2, CB), 1)
    s = jnp.where(gpos < NUM_CANDIDATES, s, NEG)
    s3 = s.reshape(2, R, 128)
    S[:, pl.ds(j * R, R), :] = s3
    RM[:, pl.ds(j * R, R), :] = jnp.max(s3, axis=-1, keepdims=True)

    @pl.when(j == GRID - 1)
    def _select():
        lane = _lane_iota()
        row_iota = lax.broadcasted_iota(jnp.int32, (NC, 128), 0)
        lane2 = lax.broadcasted_iota(jnp.int32, (NC, 128), 1)
        col_iota = lax.broadcasted_iota(jnp.int32, (NC, 1), 0)
        for qq in range(2):
            l2col = jnp.full((NC, 1), NEG, jnp.float32)
            for c in range(NC):
                col = RM[qq, pl.ds(c * 128, 128), :]
                rowmax = col.reshape(1, 128)
                L1[qq, pl.ds(c, 1), :] = rowmax
                l2col = jnp.where(col_iota == c, jnp.max(rowmax), l2col)
            L2[qq] = l2col

        def peel(k, carry):
            rs0, ri0, rs1, ri1 = carry
            res = []
            for qq, (rs, ri) in ((0, (rs0, ri0)), (1, (rs1, ri1))):
                l2col = L2[qq]
                m = jnp.max(l2col)
                c = jnp.min(jnp.where(l2col == m, col_iota, BIG))
                l1full = L1[qq]
                l1row = jnp.max(
                    jnp.where(row_iota == c, l1full, NEG), axis=0, keepdims=True
                )
                l = jnp.min(jnp.where(l1row == m, lane, BIG))
                g = c * 128 + l
                srow = S[qq, pl.ds(g, 1), :]
                li = jnp.min(jnp.where(srow == m, lane, BIG))
                gidx = g * 128 + li
                new_srow = jnp.where(lane == li, NEG, srow)
                S[qq, pl.ds(g, 1), :] = new_srow
                newg = jnp.max(new_srow)
                newl1 = jnp.where(
                    (row_iota == c) & (lane2 == l), newg, l1full
                )
                L1[qq] = newl1
                newc = jnp.max(jnp.where(row_iota == c, newl1, NEG))
                L2[qq] = jnp.where(col_iota == c, newc, l2col)
                res.append(jnp.where(lane == k, m, rs))
                res.append(jnp.where(lane == k, gidx, ri))
            return tuple(res)

        init = (
            jnp.zeros((1, 128), jnp.float32), jnp.zeros((1, 128), jnp.int32),
            jnp.zeros((1, 128), jnp.float32), jnp.zeros((1, 128), jnp.int32),
        )
        rs0, ri0, rs1, ri1 = lax.fori_loop(0, K, peel, init)
        out_s[0:1, :] = rs0[:, :K]
        out_s[1:2, :] = rs1[:, :K]
        out_i[0:1, :] = ri0[:, :K]
        out_i[1:2, :] = ri1[:, :K]


def kernel(user_id, gender, age, user_table, gender_table, age_table, candidates):
    sp = jnp.concatenate([user_id, gender, age]).astype(jnp.int32)
    ut = user_table.reshape(-1, 1, D)
    gt = gender_table.reshape(-1, 1, D)
    at = age_table.reshape(-1, 1, D)
    row = lambda k: pl.BlockSpec((1, 1, D), lambda j, sp: (sp[k], 0, 0))
    grid_spec = pltpu.PrefetchScalarGridSpec(
        num_scalar_prefetch=1,
        grid=(GRID,),
        in_specs=[
            row(0), row(1),
            row(2), row(3),
            row(4), row(5),
            pl.BlockSpec((CB, D), lambda j, sp: (j, 0)),
        ],
        out_specs=[
            pl.BlockSpec((2, K), lambda j, sp: (0, 0)),
            pl.BlockSpec((2, K), lambda j, sp: (0, 0)),
        ],
        scratch_shapes=[
            pltpu.VMEM((2, NSUB, 128), jnp.float32),
            pltpu.VMEM((2, NSUB, 1), jnp.float32),
            pltpu.VMEM((2, NC, 128), jnp.float32),
            pltpu.VMEM((2, NC, 1), jnp.float32),
        ],
    )
    out_s, out_i = pl.pallas_call(
        _body,
        grid_spec=grid_spec,
        out_shape=[
            jax.ShapeDtypeStruct((2, K), jnp.float32),
            jax.ShapeDtypeStruct((2, K), jnp.int32),
        ],
    )(sp, ut, ut, gt, gt, at, at, candidates)
    return out_s, out_i


# speedup vs baseline: 1.0126x; 1.0126x over previous
import jax
import jax.numpy as jnp
from jax import lax
from jax.experimental import pallas as pl
from jax.experimental.pallas import tpu as pltpu

NUM_CANDIDATES = 1000000
D = 64
K = 100
CB = 8192
R = CB // 128
GRID = -(-NUM_CANDIDATES // CB)
NSUB = 62 * 128
NC = 62
NEG = float("-inf")
BIG = 1 << 30


def _lane_iota():
    return lax.broadcasted_iota(jnp.int32, (1, 128), 1)


def _body(sp, u0, u1, g0, g1, a0, a1, cand, out_s, out_i, S, L1, L2):
    j = pl.program_id(0)

    @pl.when(j == 0)
    def _init_tail():
        S[:, GRID * R:, :] = jnp.full((2, NSUB - GRID * R, 128), NEG, jnp.float32)
        L1[:, GRID:, :] = jnp.full((2, 128 - GRID, R), NEG, jnp.float32)
        L2[:, GRID:, :] = jnp.full((2, 128 - GRID, 1), NEG, jnp.float32)

    q = jnp.concatenate(
        [u0[0] + g0[0] + a0[0], u1[0] + g1[0] + a1[0]], axis=0
    )
    s = lax.dot_general(
        q, cand[...], (((1,), (1,)), ((), ())),
        preferred_element_type=jnp.float32,
    )
    gpos = j * CB + lax.broadcasted_iota(jnp.int32, (2, CB), 1)
    s = jnp.where(gpos < NUM_CANDIDATES, s, NEG)
    s3 = s.reshape(2, R, 128)
    S[:, pl.ds(j * R, R), :] = s3
    rm = jnp.max(s3, axis=-1)
    L1[:, pl.ds(j, 1), :] = rm.reshape(2, 1, R)
    L2[:, pl.ds(j, 1), :] = jnp.max(rm, axis=-1, keepdims=True).reshape(2, 1, 1)

    @pl.when(j == GRID - 1)
    def _select():
        lane = _lane_iota()
        laneR = lax.broadcasted_iota(jnp.int32, (1, R), 1)

        def peel(k, carry):
            rs0, ri0, rs1, ri1, l2a, l2b = carry
            res = []
            for qq, (rs, ri, l2r) in ((0, (rs0, ri0, l2a)), (1, (rs1, ri1, l2b))):
                m = jnp.max(l2r)
                c = jnp.min(jnp.where(l2r == m, lane, BIG))
                l1row = L1[qq, pl.ds(c, 1), :]
                l = jnp.min(jnp.where(l1row == m, laneR, BIG))
                g = c * R + l
                srow = S[qq, pl.ds(g, 1), :]
                li = jnp.min(jnp.where(srow == m, lane, BIG))
                gidx = g * 128 + li
                new_srow = jnp.where(lane == li, NEG, srow)
                S[qq, pl.ds(g, 1), :] = new_srow
                newg = jnp.max(new_srow)
                newl1 = jnp.where(laneR == l, newg, l1row)
                L1[qq, pl.ds(c, 1), :] = newl1
                newc = jnp.max(newl1)
                res.append(jnp.where(lane == k, m, rs))
                res.append(jnp.where(lane == k, gidx, ri))
                res.append(jnp.where(lane == c, newc, l2r))
            return (res[0], res[1], res[3], res[4], res[2], res[5])

        init = (
            jnp.zeros((1, 128), jnp.float32), jnp.zeros((1, 128), jnp.int32),
            jnp.zeros((1, 128), jnp.float32), jnp.zeros((1, 128), jnp.int32),
            L2[0].reshape(1, 128), L2[1].reshape(1, 128),
        )
        rs0, ri0, rs1, ri1, _, _ = lax.fori_loop(0, K, peel, init)
        out_s[0:1, :] = rs0[:, :K]
        out_s[1:2, :] = rs1[:, :K]
        out_i[0:1, :] = ri0[:, :K]
        out_i[1:2, :] = ri1[:, :K]


def kernel(user_id, gender, age, user_table, gender_table, age_table, candidates):
    sp = jnp.concatenate([user_id, gender, age]).astype(jnp.int32)
    ut = user_table.reshape(-1, 1, D)
    gt = gender_table.reshape(-1, 1, D)
    at = age_table.reshape(-1, 1, D)
    row = lambda k: pl.BlockSpec((1, 1, D), lambda j, sp: (sp[k], 0, 0))
    grid_spec = pltpu.PrefetchScalarGridSpec(
        num_scalar_prefetch=1,
        grid=(GRID,),
        in_specs=[
            row(0), row(1),
            row(2), row(3),
            row(4), row(5),
            pl.BlockSpec((CB, D), lambda j, sp: (j, 0)),
        ],
        out_specs=[
            pl.BlockSpec((2, K), lambda j, sp: (0, 0)),
            pl.BlockSpec((2, K), lambda j, sp: (0, 0)),
        ],
        scratch_shapes=[
            pltpu.VMEM((2, NSUB, 128), jnp.float32),
            pltpu.VMEM((2, 128, R), jnp.float32),
            pltpu.VMEM((2, 128, 1), jnp.float32),
        ],
    )
    out_s, out_i = pl.pallas_call(
        _body,
        grid_spec=grid_spec,
        out_shape=[
            jax.ShapeDtypeStruct((2, K), jnp.float32),
            jax.ShapeDtypeStruct((2, K), jnp.int32),
        ],
    )(sp, ut, ut, gt, gt, at, at, candidates)
    return out_s, out_i


# speedup vs baseline: 1.0724x; 1.0590x over previous
import jax
import jax.numpy as jnp
from jax import lax
from jax.experimental import pallas as pl
from jax.experimental.pallas import tpu as pltpu

NUM_CANDIDATES = 1000000
D = 64
K = 100
CB = 32768
R = CB // 128
GRID = -(-NUM_CANDIDATES // CB)
NSUB = GRID * R
LPAD = 32
NEG = float("-inf")
BIG = 1 << 30


def _lane_iota():
    return lax.broadcasted_iota(jnp.int32, (1, 128), 1)


def _body(sp, u0, u1, g0, g1, a0, a1, cand, out_s, out_i, S, L1, L2):
    j = pl.program_id(0)

    @pl.when(j == 0)
    def _init_tail():
        L1[:, GRID:, :] = jnp.full((2, LPAD - GRID, R), NEG, jnp.float32)
        L2[:, GRID:, :] = jnp.full((2, LPAD - GRID, 1), NEG, jnp.float32)

    q = jnp.concatenate(
        [u0[0] + g0[0] + a0[0], u1[0] + g1[0] + a1[0]], axis=0
    )
    s = lax.dot_general(
        q, cand[...], (((1,), (1,)), ((), ())),
        preferred_element_type=jnp.float32,
    )
    gpos = j * CB + lax.broadcasted_iota(jnp.int32, (2, CB), 1)
    s = jnp.where(gpos < NUM_CANDIDATES, s, NEG)
    s3 = s.reshape(2, R, 128)
    S[:, pl.ds(j * R, R), :] = s3
    rm = jnp.max(s3, axis=-1)
    L1[:, pl.ds(j, 1), :] = rm.reshape(2, 1, R)
    L2[:, pl.ds(j, 1), :] = jnp.max(rm, axis=-1, keepdims=True).reshape(2, 1, 1)

    @pl.when(j == GRID - 1)
    def _select():
        lane = _lane_iota()
        laneR = lax.broadcasted_iota(jnp.int32, (1, R), 1)
        laneL = lax.broadcasted_iota(jnp.int32, (1, LPAD), 1)

        def peel(k, carry):
            rs0, ri0, rs1, ri1, l2a, l2b = carry
            res = []
            for qq, (rs, ri, l2r) in ((0, (rs0, ri0, l2a)), (1, (rs1, ri1, l2b))):
                m = jnp.max(l2r)
                c = jnp.min(jnp.where(l2r == m, laneL, BIG))
                l1row = L1[qq, pl.ds(c, 1), :]
                l = jnp.min(jnp.where(l1row == m, laneR, BIG))
                g = c * R + l
                srow = S[qq, pl.ds(g, 1), :]
                li = jnp.min(jnp.where(srow == m, lane, BIG))
                gidx = g * 128 + li
                new_srow = jnp.where(lane == li, NEG, srow)
                S[qq, pl.ds(g, 1), :] = new_srow
                newg = jnp.max(new_srow)
                newl1 = jnp.where(laneR == l, newg, l1row)
                L1[qq, pl.ds(c, 1), :] = newl1
                newc = jnp.max(newl1)
                res.append(jnp.where(lane == k, m, rs))
                res.append(jnp.where(lane == k, gidx, ri))
                res.append(jnp.where(laneL == c, newc, l2r))
            return (res[0], res[1], res[3], res[4], res[2], res[5])

        init = (
            jnp.zeros((1, 128), jnp.float32), jnp.zeros((1, 128), jnp.int32),
            jnp.zeros((1, 128), jnp.float32), jnp.zeros((1, 128), jnp.int32),
            L2[0].reshape(1, LPAD), L2[1].reshape(1, LPAD),
        )
        rs0, ri0, rs1, ri1, _, _ = lax.fori_loop(0, K, peel, init)
        out_s[0:1, :] = rs0[:, :K]
        out_s[1:2, :] = rs1[:, :K]
        out_i[0:1, :] = ri0[:, :K]
        out_i[1:2, :] = ri1[:, :K]


def kernel(user_id, gender, age, user_table, gender_table, age_table, candidates):
    sp = jnp.concatenate([user_id, gender, age]).astype(jnp.int32)
    ut = user_table.reshape(-1, 1, D)
    gt = gender_table.reshape(-1, 1, D)
    at = age_table.reshape(-1, 1, D)
    row = lambda k: pl.BlockSpec((1, 1, D), lambda j, sp: (sp[k], 0, 0))
    grid_spec = pltpu.PrefetchScalarGridSpec(
        num_scalar_prefetch=1,
        grid=(GRID,),
        in_specs=[
            row(0), row(1),
            row(2), row(3),
            row(4), row(5),
            pl.BlockSpec((CB, D), lambda j, sp: (j, 0)),
        ],
        out_specs=[
            pl.BlockSpec((2, K), lambda j, sp: (0, 0)),
            pl.BlockSpec((2, K), lambda j, sp: (0, 0)),
        ],
        scratch_shapes=[
            pltpu.VMEM((2, NSUB, 128), jnp.float32),
            pltpu.VMEM((2, LPAD, R), jnp.float32),
            pltpu.VMEM((2, LPAD, 1), jnp.float32),
        ],
    )
    out_s, out_i = pl.pallas_call(
        _body,
        grid_spec=grid_spec,
        out_shape=[
            jax.ShapeDtypeStruct((2, K), jnp.float32),
            jax.ShapeDtypeStruct((2, K), jnp.int32),
        ],
    )(sp, ut, ut, gt, gt, at, at, candidates)
    return out_s, out_i


# speedup vs baseline: 1.1607x; 1.0823x over previous
import jax
import jax.numpy as jnp
from jax import lax
from jax.experimental import pallas as pl
from jax.experimental.pallas import tpu as pltpu

NUM_CANDIDATES = 1000000
D = 64
K = 100
CB = 32768
R = CB // 128
GRID = -(-NUM_CANDIDATES // CB)
NSUB = GRID * R
LPAD = 32
NEG = float("-inf")
BIG = 1 << 30


def _lane_iota():
    return lax.broadcasted_iota(jnp.int32, (1, 128), 1)


def _body(sp, u0, u1, g0, g1, a0, a1, cand, out_s, out_i, S, L1, L2):
    j = pl.program_id(0)

    @pl.when(j == 0)
    def _init_tail():
        L1[:, GRID:, :] = jnp.full((2, LPAD - GRID, R), NEG, jnp.float32)
        L2[:, GRID:, :] = jnp.full((2, LPAD - GRID, 1), NEG, jnp.float32)

    q = jnp.concatenate(
        [u0[0] + g0[0] + a0[0], u1[0] + g1[0] + a1[0]], axis=0
    )
    s = lax.dot_general(
        q, cand[...], (((1,), (1,)), ((), ())),
        preferred_element_type=jnp.float32,
    )
    gpos = j * CB + lax.broadcasted_iota(jnp.int32, (2, CB), 1)
    s = jnp.where(gpos < NUM_CANDIDATES, s, NEG)
    s3 = s.reshape(2, R, 128)
    S[:, pl.ds(j * R, R), :] = s3
    rm = jnp.max(s3, axis=-1)
    L1[:, pl.ds(j, 1), :] = rm.reshape(2, 1, R)
    L2[:, pl.ds(j, 1), :] = jnp.max(rm, axis=-1, keepdims=True).reshape(2, 1, 1)

    @pl.when(j == GRID - 1)
    def _select():
        lane = _lane_iota()
        laneR = lax.broadcasted_iota(jnp.int32, (1, R), 1)
        laneL = lax.broadcasted_iota(jnp.int32, (1, LPAD), 1)

        def peel(k, carry):
            rs0, ri0, rs1, ri1, l2a, l2b = carry
            res = []
            for qq, (rs, ri, l2r) in ((0, (rs0, ri0, l2a)), (1, (rs1, ri1, l2b))):
                m = jnp.max(l2r, axis=1, keepdims=True)
                c = jnp.min(jnp.where(l2r == m, laneL, BIG))
                l1row = L1[qq, pl.ds(c, 1), :]
                l = jnp.min(jnp.where(l1row == m, laneR, BIG))
                g = c * R + l
                srow = S[qq, pl.ds(g, 1), :]
                li = jnp.min(jnp.where(srow == m, lane, BIG),
                             axis=1, keepdims=True)
                gidx = g * 128 + li
                new_srow = jnp.where(lane == li, NEG, srow)
                S[qq, pl.ds(g, 1), :] = new_srow
                newg = jnp.max(new_srow, axis=1, keepdims=True)
                newl1 = jnp.where(laneR == l, newg, l1row)
                L1[qq, pl.ds(c, 1), :] = newl1
                newc = jnp.max(newl1, axis=1, keepdims=True)
                res.append(jnp.where(lane == k, m, rs))
                res.append(jnp.where(lane == k, gidx, ri))
                res.append(jnp.where(laneL == c, newc, l2r))
            return (res[0], res[1], res[3], res[4], res[2], res[5])

        init = (
            jnp.zeros((1, 128), jnp.float32), jnp.zeros((1, 128), jnp.int32),
            jnp.zeros((1, 128), jnp.float32), jnp.zeros((1, 128), jnp.int32),
            L2[0].reshape(1, LPAD), L2[1].reshape(1, LPAD),
        )
        rs0, ri0, rs1, ri1, _, _ = lax.fori_loop(0, K, peel, init)
        out_s[0:1, :] = rs0[:, :K]
        out_s[1:2, :] = rs1[:, :K]
        out_i[0:1, :] = ri0[:, :K]
        out_i[1:2, :] = ri1[:, :K]


def kernel(user_id, gender, age, user_table, gender_table, age_table, candidates):
    sp = jnp.concatenate([user_id, gender, age]).astype(jnp.int32)
    ut = user_table.reshape(-1, 1, D)
    gt = gender_table.reshape(-1, 1, D)
    at = age_table.reshape(-1, 1, D)
    row = lambda k: pl.BlockSpec((1, 1, D), lambda j, sp: (sp[k], 0, 0))
    grid_spec = pltpu.PrefetchScalarGridSpec(
        num_scalar_prefetch=1,
        grid=(GRID,),
        in_specs=[
            row(0), row(1),
            row(2), row(3),
            row(4), row(5),
            pl.BlockSpec((CB, D), lambda j, sp: (j, 0)),
        ],
        out_specs=[
            pl.BlockSpec((2, K), lambda j, sp: (0, 0)),
            pl.BlockSpec((2, K), lambda j, sp: (0, 0)),
        ],
        scratch_shapes=[
            pltpu.VMEM((2, NSUB, 128), jnp.float32),
            pltpu.VMEM((2, LPAD, R), jnp.float32),
            pltpu.VMEM((2, LPAD, 1), jnp.float32),
        ],
    )
    out_s, out_i = pl.pallas_call(
        _body,
        grid_spec=grid_spec,
        out_shape=[
            jax.ShapeDtypeStruct((2, K), jnp.float32),
            jax.ShapeDtypeStruct((2, K), jnp.int32),
        ],
    )(sp, ut, ut, gt, gt, at, at, candidates)
    return out_s, out_i
